# baseline (speedup 1.0000x reference)
import functools

import jax
import jax.numpy as jnp
from jax import lax
from jax.experimental import pallas as pl
from jax.experimental.pallas import tpu as pltpu
from jax.experimental.pallas import tpu_sc as plsc


def _make_gather(V, D, B):
    info = plsc.get_sparse_core_info()
    NC, NS = info.num_cores, info.num_subcores
    NW = NC * NS
    assert B % NW == 0
    b_per_w = B // NW
    C = 48
    NB = 2
    chunks = []
    off = 0
    while off < b_per_w:
        n = min(C, b_per_w - off)
        chunks.append((off, n))
        off += n
    n_chunks = len(chunks)

    mesh = plsc.VectorSubcoreMesh(core_axis_name="c", subcore_axis_name="s")

    @functools.partial(
        pl.kernel,
        out_type=jax.ShapeDtypeStruct((B, D), jnp.float32),
        mesh=mesh,
        scratch_types=[
            pltpu.VMEM((b_per_w,), jnp.int32),
            *[pltpu.VMEM((C, D), jnp.float32) for _ in range(NB)],
            *[pltpu.SemaphoreType.DMA for _ in range(2 * NB)],
        ],
    )
    def gather_kernel(table_hbm, idx_hbm, out_hbm, idx_v, *bufs_and_sems):
        bufs = bufs_and_sems[:NB]
        gsem = bufs_and_sems[NB : 2 * NB]
        wsem = bufs_and_sems[2 * NB :]
        wid = lax.axis_index("s") * NC + lax.axis_index("c")
        base = wid * b_per_w
        pltpu.sync_copy(idx_hbm.at[pl.ds(base, b_per_w)], idx_v)

        def gather(g):
            b = g % NB
            off, n = chunks[g]
            return pltpu.async_copy(
                table_hbm.at[idx_v.at[pl.ds(off, n)]],
                bufs[b].at[pl.ds(0, n)],
                gsem[b],
            )

        rd = {0: gather(0)}
        wr = {}
        for g in range(n_chunks):
            b = g % NB
            off, n = chunks[g]
            rd[g].wait()
            wr[g] = pltpu.async_copy(
                bufs[b].at[pl.ds(0, n)],
                out_hbm.at[pl.ds(base + off, n)],
                wsem[b],
            )
            if g + 1 < n_chunks:
                if g - 1 >= 0:
                    wr[g - 1].wait()
                rd[g + 1] = gather(g + 1)
        wr[n_chunks - 2].wait()
        wr[n_chunks - 1].wait()


    return gather_kernel


def kernel(positional_encoding, t):
    V, D = positional_encoding.shape
    (B,) = t.shape
    gather = _make_gather(V, D, B)
    return gather(positional_encoding, t.astype(jnp.int32))


# speedup vs baseline: 1.0518x; 1.0518x over previous
import functools

import jax
import jax.numpy as jnp
from jax import lax
from jax.experimental import pallas as pl
from jax.experimental.pallas import tpu as pltpu
from jax.experimental.pallas import tpu_sc as plsc


def _make_gather(V, D, B):
    info = plsc.get_sparse_core_info()
    NC, NS = info.num_cores, info.num_subcores
    NW = NC * NS
    assert B % NW == 0
    b_per_w = B // NW
    C = 16
    NB = 6
    n_chunks = b_per_w // C
    assert b_per_w % C == 0

    mesh = plsc.VectorSubcoreMesh(core_axis_name="c", subcore_axis_name="s")

    @functools.partial(
        pl.kernel,
        out_type=jax.ShapeDtypeStruct((B, D), jnp.float32),
        mesh=mesh,
        scratch_types=[
            pltpu.VMEM((b_per_w,), jnp.int32),
            *[pltpu.VMEM((C, D), jnp.float32) for _ in range(NB)],
            *[pltpu.SemaphoreType.DMA for _ in range(2 * NB)],
        ],
    )
    def gather_kernel(table_hbm, idx_hbm, out_hbm, idx_v, *bufs_and_sems):
        bufs = bufs_and_sems[:NB]
        gsem = bufs_and_sems[NB : 2 * NB]
        wsem = bufs_and_sems[2 * NB :]
        wid = lax.axis_index("s") * NC + lax.axis_index("c")
        base = wid * b_per_w
        pltpu.sync_copy(idx_hbm.at[pl.ds(base, b_per_w)], idx_v)

        def gather(g):
            b = g % NB
            return pltpu.async_copy(
                table_hbm.at[idx_v.at[pl.ds(g * C, C)]], bufs[b], gsem[b]
            )

        rd = {g: gather(g) for g in range(NB - 1)}
        wr = {}
        for g in range(n_chunks):
            b = g % NB
            rd[g].wait()
            wr[g] = pltpu.async_copy(
                bufs[b], out_hbm.at[pl.ds(base + g * C, C)], wsem[b]
            )
            if g + NB - 1 < n_chunks:
                if g - 1 >= 0:
                    wr[g - 1].wait()
                rd[g + NB - 1] = gather(g + NB - 1)
        for g in range(n_chunks - NB + 1, n_chunks):
            wr[g - 1].wait()
        wr[n_chunks - 1].wait()

    return gather_kernel


def kernel(positional_encoding, t):
    V, D = positional_encoding.shape
    (B,) = t.shape
    gather = _make_gather(V, D, B)
    return gather(positional_encoding, t.astype(jnp.int32))
